# Initial kernel scaffold; baseline (speedup 1.0000x reference)
import jax
import jax.numpy as jnp
from jax.experimental import pallas as pl


def kernel(x, W1, b1, W2, b2, W3, b3, W4, b4):
    raise NotImplementedError("write your pallas kernel here")


# speedup vs baseline: 5.0004x; 5.0004x over previous
import jax
import jax.numpy as jnp
from jax.experimental import pallas as pl
from jax.experimental.pallas import tpu as pltpu

_MD = 2048
_HID = 1024
_TILE = 512
_SUB = 2


def _gelu(v):
    return 0.5 * v * (1.0 + jax.lax.erf(v * 0.7071067811865476))


def _ffn_body(x_ref, w1_ref, b1_ref, w2_ref, b2_ref, w3_ref, b3_ref,
              w4_ref, b4_ref, o_ref):
    m = _TILE // _SUB
    sub = range(_SUB)
    xb = [x_ref[pl.ds(s * m, m), :].astype(jnp.bfloat16) for s in sub]
    h = [jnp.dot(xb[s], w1_ref[...], preferred_element_type=jnp.float32)
         for s in sub]
    h = [_gelu(h[s] + b1_ref[...]).astype(jnp.bfloat16) for s in sub]
    y = [jnp.dot(h[s], w2_ref[...], preferred_element_type=jnp.float32)
         for s in sub]
    y = [(y[s] + b2_ref[...]).astype(jnp.bfloat16) for s in sub]
    g = [jnp.dot(y[s], w3_ref[...], preferred_element_type=jnp.float32)
         for s in sub]
    g = [_gelu(g[s] + b3_ref[...]).astype(jnp.bfloat16) for s in sub]
    o = [jnp.dot(g[s], w4_ref[...], preferred_element_type=jnp.float32)
         for s in sub]
    for s in sub:
        o_ref[pl.ds(s * m, m), :] = o[s] + b4_ref[...]


def kernel(x, W1, b1, W2, b2, W3, b3, W4, b4):
    B, S, D = x.shape
    N = B * S
    xf = x.reshape(N, D)
    w1 = W1.astype(jnp.bfloat16)
    w2 = W2.astype(jnp.bfloat16)
    w3 = W3.astype(jnp.bfloat16)
    w4 = W4.astype(jnp.bfloat16)
    b1r = b1.reshape(1, _HID)
    b2r = b2.reshape(1, _MD)
    b3r = b3.reshape(1, _MD)
    b4r = b4.reshape(1, _MD)

    vmem = pl.BlockSpec(memory_space=pltpu.VMEM)
    out = pl.pallas_call(
        _ffn_body,
        grid=(N // _TILE,),
        in_specs=[
            pl.BlockSpec((_TILE, _MD), lambda i: (i, 0)),
            vmem, vmem, vmem, vmem, vmem, vmem, vmem, vmem,
        ],
        out_specs=pl.BlockSpec((_TILE, _MD), lambda i: (i, 0)),
        out_shape=jax.ShapeDtypeStruct((N, _MD), jnp.float32),
        compiler_params=pltpu.CompilerParams(
            dimension_semantics=("arbitrary",),
        ),
    )(xf, w1, b1r, w2, b2r, w3, b3r, w4, b4r)
    return out.reshape(B, S, D)


# speedup vs baseline: 5.1873x; 1.0374x over previous
import jax
import jax.numpy as jnp
from jax.experimental import pallas as pl
from jax.experimental.pallas import tpu as pltpu

_MD = 2048
_HID = 1024
_TILE = 128


def _gelu(v):
    return 0.5 * v * (1.0 + jax.lax.erf(v * 0.7071067811865476))


def _ffn_body(x_ref, w1_ref, b1_ref, w2_ref, b2_ref, w3_ref, b3_ref,
              w4_ref, b4_ref, o_ref):
    h = jnp.dot(x_ref[...], w1_ref[...], preferred_element_type=jnp.float32)
    h = _gelu(h + b1_ref[...])
    y = jnp.dot(h, w2_ref[...], preferred_element_type=jnp.float32)
    y = y + b2_ref[...]
    g = jnp.dot(y, w3_ref[...], preferred_element_type=jnp.float32)
    g = _gelu(g + b3_ref[...])
    o = jnp.dot(g, w4_ref[...], preferred_element_type=jnp.float32)
    o_ref[...] = o + b4_ref[...]


def kernel(x, W1, b1, W2, b2, W3, b3, W4, b4):
    B, S, D = x.shape
    N = B * S
    xf = x.reshape(N, D)
    b1r = b1.reshape(1, _HID)
    b2r = b2.reshape(1, _MD)
    b3r = b3.reshape(1, _MD)
    b4r = b4.reshape(1, _MD)

    vmem = pl.BlockSpec(memory_space=pltpu.VMEM)
    out = pl.pallas_call(
        _ffn_body,
        grid=(N // _TILE,),
        in_specs=[
            pl.BlockSpec((_TILE, _MD), lambda i: (i, 0)),
            vmem, vmem, vmem, vmem, vmem, vmem, vmem, vmem,
        ],
        out_specs=pl.BlockSpec((_TILE, _MD), lambda i: (i, 0)),
        out_shape=jax.ShapeDtypeStruct((N, _MD), jnp.float32),
        compiler_params=pltpu.CompilerParams(
            dimension_semantics=("arbitrary",),
        ),
    )(xf, W1, b1r, W2, b2r, W3, b3r, W4, b4r)
    return out.reshape(B, S, D)
